# baseline (speedup 1.0000x reference)
import jax
import jax.numpy as jnp
from jax.experimental import pallas as pl
from jax.experimental.pallas import tpu as pltpu


def _copy_scatter(p1_ref, p2_ref, k1_ref, k2_ref, q1_ref, q2_ref,
                  o1_ref, o2_ref, np1_ref, np2_ref,
                  sem1, sem2, sem3, sem4):
    kq = q1_ref.shape[0]
    b = k1_ref.shape[0]
    c1 = pltpu.make_async_copy(q1_ref, o1_ref, sem1)
    c2 = pltpu.make_async_copy(q2_ref, o2_ref, sem2)
    c1.start()
    c2.start()
    p1 = p1_ref[0]
    p2 = p2_ref[0]
    np1_ref[0] = jax.lax.rem(p1 + b, kq)
    np2_ref[0] = jax.lax.rem(p2 + b, kq)
    s1 = jnp.minimum(p1, kq - b)
    s2 = jnp.minimum(p2, kq - b)
    c1.wait()
    c2.wait()
    w1 = pltpu.make_async_copy(k1_ref, o1_ref.at[pl.ds(s1, b), :], sem3)
    w2 = pltpu.make_async_copy(k2_ref, o2_ref.at[pl.ds(s2, b), :], sem4)
    w1.start()
    w2.start()
    w1.wait()
    w2.wait()


def kernel(keys_1, keys_2, queue_1, queue_2, queue_1_ptr, queue_2_ptr):
    kq, d1 = queue_1.shape
    d2 = queue_2.shape[1]
    out = pl.pallas_call(
        _copy_scatter,
        in_specs=[
            pl.BlockSpec(memory_space=pltpu.SMEM),
            pl.BlockSpec(memory_space=pltpu.SMEM),
            pl.BlockSpec(memory_space=pl.ANY),
            pl.BlockSpec(memory_space=pl.ANY),
            pl.BlockSpec(memory_space=pl.ANY),
            pl.BlockSpec(memory_space=pl.ANY),
        ],
        out_specs=[
            pl.BlockSpec(memory_space=pl.ANY),
            pl.BlockSpec(memory_space=pl.ANY),
            pl.BlockSpec(memory_space=pltpu.SMEM),
            pl.BlockSpec(memory_space=pltpu.SMEM),
        ],
        out_shape=[
            jax.ShapeDtypeStruct((kq, d1), queue_1.dtype),
            jax.ShapeDtypeStruct((kq, d2), queue_2.dtype),
            jax.ShapeDtypeStruct((1,), jnp.int32),
            jax.ShapeDtypeStruct((1,), jnp.int32),
        ],
        scratch_shapes=[pltpu.SemaphoreType.DMA] * 4,
    )(queue_1_ptr, queue_2_ptr, keys_1, keys_2, queue_1, queue_2)
    return out[0], out[1], out[2], out[3]


# speedup vs baseline: 1.0000x; 1.0000x over previous
import jax
import jax.numpy as jnp
from jax.experimental import pallas as pl
from jax.experimental.pallas import tpu as pltpu


_NCHUNK = 32


def _copy_scatter(p1_ref, p2_ref, k1_ref, k2_ref, q1_ref, q2_ref,
                  o1_ref, o2_ref, np1_ref, np2_ref,
                  sem1, sem2, sem3, sem4):
    kq = q1_ref.shape[0]
    b = k1_ref.shape[0]
    ch = kq // _NCHUNK
    copies = []
    for i in range(_NCHUNK):
        sl = pl.ds(i * ch, ch)
        copies.append(pltpu.make_async_copy(q1_ref.at[sl, :], o1_ref.at[sl, :], sem1))
        copies.append(pltpu.make_async_copy(q2_ref.at[sl, :], o2_ref.at[sl, :], sem2))
    for c in copies:
        c.start()
    p1 = p1_ref[0]
    p2 = p2_ref[0]
    np1_ref[0] = jax.lax.rem(p1 + b, kq)
    np2_ref[0] = jax.lax.rem(p2 + b, kq)
    s1 = jnp.minimum(p1, kq - b)
    s2 = jnp.minimum(p2, kq - b)
    for c in copies:
        c.wait()
    w1 = pltpu.make_async_copy(k1_ref, o1_ref.at[pl.ds(s1, b), :], sem3)
    w2 = pltpu.make_async_copy(k2_ref, o2_ref.at[pl.ds(s2, b), :], sem4)
    w1.start()
    w2.start()
    w1.wait()
    w2.wait()


def kernel(keys_1, keys_2, queue_1, queue_2, queue_1_ptr, queue_2_ptr):
    kq, d1 = queue_1.shape
    d2 = queue_2.shape[1]
    out = pl.pallas_call(
        _copy_scatter,
        in_specs=[
            pl.BlockSpec(memory_space=pltpu.SMEM),
            pl.BlockSpec(memory_space=pltpu.SMEM),
            pl.BlockSpec(memory_space=pl.ANY),
            pl.BlockSpec(memory_space=pl.ANY),
            pl.BlockSpec(memory_space=pl.ANY),
            pl.BlockSpec(memory_space=pl.ANY),
        ],
        out_specs=[
            pl.BlockSpec(memory_space=pl.ANY),
            pl.BlockSpec(memory_space=pl.ANY),
            pl.BlockSpec(memory_space=pltpu.SMEM),
            pl.BlockSpec(memory_space=pltpu.SMEM),
        ],
        out_shape=[
            jax.ShapeDtypeStruct((kq, d1), queue_1.dtype),
            jax.ShapeDtypeStruct((kq, d2), queue_2.dtype),
            jax.ShapeDtypeStruct((1,), jnp.int32),
            jax.ShapeDtypeStruct((1,), jnp.int32),
        ],
        scratch_shapes=[pltpu.SemaphoreType.DMA] * 4,
    )(queue_1_ptr, queue_2_ptr, keys_1, keys_2, queue_1, queue_2)
    return out[0], out[1], out[2], out[3]


# speedup vs baseline: 18.5277x; 18.5269x over previous
import jax
import jax.numpy as jnp
from jax.experimental import pallas as pl
from jax.experimental.pallas import tpu as pltpu

_R = 8000


def _blend(i, p_ref, kp_ref, q_ref, o_ref, r, b, kq):
    p = p_ref[0]
    s_raw = i * r - p
    s = jnp.where(s_raw < -r, s_raw + kq, s_raw)

    @pl.when(s >= b)
    def _copy():
        o_ref[...] = q_ref[...]

    @pl.when(s < b)
    def _window():
        d = q_ref.shape[1]
        shifted = kp_ref[pl.ds(s + r, r), :]
        rows = jax.lax.broadcasted_iota(jnp.int32, (r, d), 0) + s
        mask = (rows >= 0) & (rows < b)
        o_ref[...] = jnp.where(mask, shifted, q_ref[...])


def _body(p1_ref, p2_ref, kp1_ref, kp2_ref, q1_ref, q2_ref,
          o1_ref, o2_ref, np1_ref, np2_ref, *, b, kq):
    i = pl.program_id(0)
    _blend(i, p1_ref, kp1_ref, q1_ref, o1_ref, _R, b, kq)
    _blend(i, p2_ref, kp2_ref, q2_ref, o2_ref, _R, b, kq)

    @pl.when(i == 0)
    def _ptrs():
        np1_ref[0] = jax.lax.rem(p1_ref[0] + b, kq)
        np2_ref[0] = jax.lax.rem(p2_ref[0] + b, kq)


def kernel(keys_1, keys_2, queue_1, queue_2, queue_1_ptr, queue_2_ptr):
    kq, d1 = queue_1.shape
    d2 = queue_2.shape[1]
    b = keys_1.shape[0]
    assert kq % _R == 0
    steps = kq // _R

    kp1 = jnp.pad(keys_1, ((_R, _R), (0, 0)))
    kp2 = jnp.pad(keys_2, ((_R, _R), (0, 0)))

    import functools
    out = pl.pallas_call(
        functools.partial(_body, b=b, kq=kq),
        grid=(steps,),
        in_specs=[
            pl.BlockSpec(memory_space=pltpu.SMEM),
            pl.BlockSpec(memory_space=pltpu.SMEM),
            pl.BlockSpec((b + 2 * _R, d1), lambda i: (0, 0)),
            pl.BlockSpec((b + 2 * _R, d2), lambda i: (0, 0)),
            pl.BlockSpec((_R, d1), lambda i: (i, 0)),
            pl.BlockSpec((_R, d2), lambda i: (i, 0)),
        ],
        out_specs=[
            pl.BlockSpec((_R, d1), lambda i: (i, 0)),
            pl.BlockSpec((_R, d2), lambda i: (i, 0)),
            pl.BlockSpec(memory_space=pltpu.SMEM),
            pl.BlockSpec(memory_space=pltpu.SMEM),
        ],
        out_shape=[
            jax.ShapeDtypeStruct((kq, d1), queue_1.dtype),
            jax.ShapeDtypeStruct((kq, d2), queue_2.dtype),
            jax.ShapeDtypeStruct((1,), jnp.int32),
            jax.ShapeDtypeStruct((1,), jnp.int32),
        ],
    )(queue_1_ptr, queue_2_ptr, kp1, kp2, queue_1, queue_2)
    return out[0], out[1], out[2], out[3]


# speedup vs baseline: 205.9799x; 11.1174x over previous
import functools

import jax
import jax.numpy as jnp
from jax.experimental import pallas as pl
from jax.experimental.pallas import tpu as pltpu

_C = 8192


def _blend(i, p_ref, kp_ref, q_ref, o_ref, c, b, kq):
    p = p_ref[0]
    s_raw = i * c - p
    s = jnp.where(s_raw < -c, s_raw + kq, s_raw)

    @pl.when(s >= b)
    def _copy():
        o_ref[...] = q_ref[...]

    @pl.when(s < b)
    def _window():
        d = q_ref.shape[0]
        u = s + c
        u128 = jnp.floor_divide(u, 128) * 128
        r = u - u128
        ext = kp_ref[:, pl.ds(pl.multiple_of(u128, 128), c + 128)]
        rolled = pltpu.roll(ext, (c + 128) - r, 1)
        shifted = rolled[:, :c]
        cols = jax.lax.broadcasted_iota(jnp.int32, (d, c), 1) + s
        mask = (cols >= 0) & (cols < b)
        o_ref[...] = jnp.where(mask, shifted, q_ref[...])


def _body(p1_ref, p2_ref, kp1_ref, kp2_ref, q1_ref, q2_ref,
          o1_ref, o2_ref, np1_ref, np2_ref, *, b, kq):
    i = pl.program_id(0)
    _blend(i, p1_ref, kp1_ref, q1_ref, o1_ref, _C, b, kq)
    _blend(i, p2_ref, kp2_ref, q2_ref, o2_ref, _C, b, kq)

    @pl.when(i == 0)
    def _ptrs():
        np1_ref[0] = jax.lax.rem(p1_ref[0] + b, kq)
        np2_ref[0] = jax.lax.rem(p2_ref[0] + b, kq)


def kernel(keys_1, keys_2, queue_1, queue_2, queue_1_ptr, queue_2_ptr):
    kq, d1 = queue_1.shape
    d2 = queue_2.shape[1]
    b = keys_1.shape[0]
    steps = pl.cdiv(kq, _C)

    q1t = queue_1.T
    q2t = queue_2.T
    kp1 = jnp.pad(keys_1.T, ((0, 0), (_C, _C + 128)))
    kp2 = jnp.pad(keys_2.T, ((0, 0), (_C, _C + 128)))

    out = pl.pallas_call(
        functools.partial(_body, b=b, kq=kq),
        grid=(steps,),
        in_specs=[
            pl.BlockSpec(memory_space=pltpu.SMEM),
            pl.BlockSpec(memory_space=pltpu.SMEM),
            pl.BlockSpec((d1, b + 2 * _C + 128), lambda i: (0, 0)),
            pl.BlockSpec((d2, b + 2 * _C + 128), lambda i: (0, 0)),
            pl.BlockSpec((d1, _C), lambda i: (0, i)),
            pl.BlockSpec((d2, _C), lambda i: (0, i)),
        ],
        out_specs=[
            pl.BlockSpec((d1, _C), lambda i: (0, i)),
            pl.BlockSpec((d2, _C), lambda i: (0, i)),
            pl.BlockSpec(memory_space=pltpu.SMEM),
            pl.BlockSpec(memory_space=pltpu.SMEM),
        ],
        out_shape=[
            jax.ShapeDtypeStruct((d1, kq), queue_1.dtype),
            jax.ShapeDtypeStruct((d2, kq), queue_2.dtype),
            jax.ShapeDtypeStruct((1,), jnp.int32),
            jax.ShapeDtypeStruct((1,), jnp.int32),
        ],
    )(queue_1_ptr, queue_2_ptr, kp1, kp2, q1t, q2t)
    return out[0].T, out[1].T, out[2], out[3]


# speedup vs baseline: 235.7502x; 1.1445x over previous
import functools

import jax
import jax.numpy as jnp
from jax.experimental import pallas as pl
from jax.experimental.pallas import tpu as pltpu

_C = 16384


def _blend(i, p_ref, kp_ref, q_ref, o_ref, c, b, kq):
    p = p_ref[0]
    s_raw = i * c - p
    s = jnp.where(s_raw < -c, s_raw + kq, s_raw)

    @pl.when(s >= b)
    def _copy():
        o_ref[...] = q_ref[...]

    @pl.when(s < b)
    def _window():
        d = q_ref.shape[0]
        u = s + c
        u128 = jnp.floor_divide(u, 128) * 128
        r = u - u128
        ext = kp_ref[:, pl.ds(pl.multiple_of(u128, 128), c + 128)]
        rolled = pltpu.roll(ext, (c + 128) - r, 1)
        shifted = rolled[:, :c]
        cols = jax.lax.broadcasted_iota(jnp.int32, (d, c), 1) + s
        mask = (cols >= 0) & (cols < b)
        o_ref[...] = jnp.where(mask, shifted, q_ref[...])


def _body(p1_ref, p2_ref, kp1_ref, kp2_ref, q1_ref, q2_ref,
          o1_ref, o2_ref, np1_ref, np2_ref, *, b, kq):
    i = pl.program_id(0)
    _blend(i, p1_ref, kp1_ref, q1_ref, o1_ref, _C, b, kq)
    _blend(i, p2_ref, kp2_ref, q2_ref, o2_ref, _C, b, kq)

    @pl.when(i == 0)
    def _ptrs():
        np1_ref[0] = jax.lax.rem(p1_ref[0] + b, kq)
        np2_ref[0] = jax.lax.rem(p2_ref[0] + b, kq)


def kernel(keys_1, keys_2, queue_1, queue_2, queue_1_ptr, queue_2_ptr):
    kq, d1 = queue_1.shape
    d2 = queue_2.shape[1]
    b = keys_1.shape[0]
    steps = pl.cdiv(kq, _C)

    q1t = queue_1.T
    q2t = queue_2.T
    kp1 = jnp.pad(keys_1.T, ((0, 0), (_C, _C + 128)))
    kp2 = jnp.pad(keys_2.T, ((0, 0), (_C, _C + 128)))

    out = pl.pallas_call(
        functools.partial(_body, b=b, kq=kq),
        grid=(steps,),
        in_specs=[
            pl.BlockSpec(memory_space=pltpu.SMEM),
            pl.BlockSpec(memory_space=pltpu.SMEM),
            pl.BlockSpec((d1, b + 2 * _C + 128), lambda i: (0, 0)),
            pl.BlockSpec((d2, b + 2 * _C + 128), lambda i: (0, 0)),
            pl.BlockSpec((d1, _C), lambda i: (0, i)),
            pl.BlockSpec((d2, _C), lambda i: (0, i)),
        ],
        out_specs=[
            pl.BlockSpec((d1, _C), lambda i: (0, i)),
            pl.BlockSpec((d2, _C), lambda i: (0, i)),
            pl.BlockSpec(memory_space=pltpu.SMEM),
            pl.BlockSpec(memory_space=pltpu.SMEM),
        ],
        out_shape=[
            jax.ShapeDtypeStruct((d1, kq), queue_1.dtype),
            jax.ShapeDtypeStruct((d2, kq), queue_2.dtype),
            jax.ShapeDtypeStruct((1,), jnp.int32),
            jax.ShapeDtypeStruct((1,), jnp.int32),
        ],
    )(queue_1_ptr, queue_2_ptr, kp1, kp2, q1t, q2t)
    return out[0].T, out[1].T, out[2], out[3]
